# baseline (speedup 1.0000x reference)
import jax
import jax.numpy as jnp
from jax.experimental import pallas as pl

F_SIZE = 30
K_HEADS = 2
NUM_GRAPHS = 64


def _sat(p, feats, idx, n):
    h = feats @ p['W'] + p['b']
    a1 = (h @ p['a1_W'] + p['a1_b'])[:, 0]
    a2 = (h @ p['a2_W'] + p['a2_b'])[:, 0]
    row = idx[0]
    col = idx[1]
    v = a1[row] + a2[col]
    v = jnp.where(v >= 0, v, 0.01 * v)
    m = jax.ops.segment_max(v, row, num_segments=n)
    ex = jnp.exp(v - m[row])
    s = jax.ops.segment_sum(ex, row, num_segments=n)
    attn = ex / s[row]
    return jnp.zeros((n, h.shape[1]), h.dtype).at[row].add(attn[:, None] * h[col])


def _pool(x, bidx, B):
    sums = jax.ops.segment_sum(x, bidx, num_segments=B)
    cnt = jax.ops.segment_sum(jnp.ones((x.shape[0],), x.dtype), bidx, num_segments=B)
    return sums / jnp.clip(cnt, 1.0)[:, None]


def _branch(params, pref, X, idx_list, bidx):
    n = X.shape[0]
    def layer(pkey, inp):
        return jax.nn.relu(jnp.concatenate([
            _sat(p, inp, idx_list[h], n)
            for h, p in enumerate(params[pkey])], axis=1))
    x1 = layer(pref + '_1', X)
    x2 = layer(pref + '_2', x1)
    x3 = layer(pref + '_3', x2)
    x4 = jnp.concatenate([x1, x2, x3], axis=1) @ params[pref + '_4W'] + params[pref + '_4b']
    return _pool(x4, bidx, NUM_GRAPHS)


def _final_kernel(x_ref, w_ref, b_ref, o_ref):
    y = jnp.dot(x_ref[...], w_ref[...], preferred_element_type=jnp.float32)
    y = y + b_ref[...][None, :]
    m = jnp.max(y, axis=1, keepdims=True)
    e = jnp.exp(y - m)
    o_ref[...] = e / jnp.sum(e, axis=1, keepdims=True)


def kernel(X0, X1, X2, L0_indices, L0_values, L1u_indices, L1u_values,
           L1d_indices, L1d_values, L2_indices, L2_values,
           batch0, batch1, batch2, params):
    x0 = _branch(params, 'l0', X0, [L0_indices, L0_indices], batch0)
    x1 = _branch(params, 'l1', X1, [L1u_indices, L1d_indices], batch1)
    x2 = _branch(params, 'l2', X2, [L2_indices, L2_indices], batch2)
    x = jnp.concatenate([x0, x1, x2], axis=1)
    return pl.pallas_call(
        _final_kernel,
        out_shape=jax.ShapeDtypeStruct((NUM_GRAPHS, 64), jnp.float32),
    )(x, params['comb_W'], params['comb_b'])


# speedup vs baseline: 35.4378x; 35.4378x over previous
import functools

import jax
import jax.numpy as jnp
from jax import lax
from jax.experimental import pallas as pl
from jax.experimental.pallas import tpu as pltpu
from jax.experimental.pallas import tpu_sc as plsc

NUM_GRAPHS = 64
CH = 80
ZR = 25
BN = 2000


@functools.lru_cache(maxsize=None)
def _make_edge(n, E):
    per_w = E // 32
    nchunks = per_w // CH
    nr = n // 16
    assert per_w % CH == 0 and n % (16 * ZR) == 0

    mesh = plsc.VectorSubcoreMesh(core_axis_name="c", subcore_axis_name="s")

    @functools.partial(
        pl.kernel, mesh=mesh,
        out_type=jax.ShapeDtypeStruct((2, n, 16), jnp.float32),
        compiler_params=pltpu.CompilerParams(needs_layout_passes=False,
                                             use_tc_tiling_on_sc=False),
        scratch_types=[
            pltpu.VMEM((n,), jnp.float32),
            pltpu.VMEM((n,), jnp.float32),
            pltpu.VMEM((CH,), jnp.int32),
            pltpu.VMEM((CH,), jnp.int32),
            pltpu.VMEM((CH, 16), jnp.float32),
            pltpu.VMEM((CH, 16), jnp.float32),
            pltpu.VMEM((ZR, 16), jnp.float32),
            pltpu.VMEM_SHARED((n, 16), jnp.float32),
            pltpu.SemaphoreType.DMA,
        ],
    )
    def edge(row_hbm, col_hbm, a1_hbm, a2_hbm, h_hbm, out_hbm,
             a1_t, a2_t, row_t, col_t, gath, contrib, zbuf, acc_sh, sem):
        c = lax.axis_index("c")
        s = lax.axis_index("s")
        wid = c * 16 + s

        pltpu.sync_copy(a1_hbm, a1_t)
        pltpu.sync_copy(a2_hbm, a2_t)

        zero16 = jnp.zeros((16,), jnp.float32)
        for i in range(ZR):
            zbuf[i] = zero16

        def zacc(k, carry):
            pltpu.sync_copy(zbuf, acc_sh.at[pl.ds(s * nr + k * ZR, ZR)])
            return carry
        lax.fori_loop(0, nr // ZR, zacc, 0)
        plsc.subcore_barrier()

        lane = lax.iota(jnp.int32, 16)

        def chunk(k, carry):
            base = wid * per_w + k * CH
            pltpu.sync_copy(row_hbm.at[pl.ds(base, CH)], row_t)
            pltpu.sync_copy(col_hbm.at[pl.ds(base, CH)], col_t)
            pltpu.async_copy(h_hbm.at[col_t], gath, sem).wait()
            for j in range(CH // 16):
                r = row_t[pl.ds(j * 16, 16)]
                cc = col_t[pl.ds(j * 16, 16)]
                v = plsc.load_gather(a1_t, [r]) + plsc.load_gather(a2_t, [cc])
                v = jnp.where(v >= 0.0, v, 0.01 * v)
                w = jnp.exp(v)
                ridx = lane + (j * 16)
                for f in range(16):
                    fidx = jnp.full((16,), f, jnp.int32)
                    hv = plsc.load_gather(gath, [ridx, fidx])
                    plsc.store_scatter(contrib, [ridx, fidx], hv * w)
            pltpu.sync_copy(contrib, acc_sh.at[row_t], add=True)
            return carry
        lax.fori_loop(0, nchunks, chunk, 0)

        plsc.subcore_barrier()

        @pl.when(s == 0)
        def _():
            pltpu.sync_copy(acc_sh, out_hbm.at[c])

    return edge


def _d1_body(x_ref, w_ref, b_ref, o_ref):
    o_ref[...] = jnp.dot(x_ref[...], w_ref[...],
                         preferred_element_type=jnp.float32) + b_ref[...]


@functools.lru_cache(maxsize=None)
def _make_d1(n, fin):
    grid = n // BN
    return pl.pallas_call(
        _d1_body,
        grid=(grid,),
        in_specs=[
            pl.BlockSpec((BN, fin), lambda i: (i, 0)),
            pl.BlockSpec((fin, 64), lambda i: (0, 0)),
            pl.BlockSpec((1, 64), lambda i: (0, 0)),
        ],
        out_specs=pl.BlockSpec((BN, 64), lambda i: (i, 0)),
        out_shape=jax.ShapeDtypeStruct((n, 64), jnp.float32),
    )


def _normalize_x(p0_ref, p1_ref):
    p0 = p0_ref[0] + p0_ref[1]
    p1 = p1_ref[0] + p1_ref[1]
    s0 = p0[:, 15:16]
    s1 = p1[:, 15:16]
    o0 = p0[:, 0:15] / jnp.where(s0 == 0.0, 1.0, s0)
    o1 = p1[:, 0:15] / jnp.where(s1 == 0.0, 1.0, s1)
    bn = o0.shape[0]
    x = jnp.concatenate([o0, o1, jnp.zeros((bn, 2), jnp.float32)], axis=1)
    return jnp.maximum(x, 0.0)


def _d2_body(p0_ref, p1_ref, w_ref, b_ref, o_ref, x_ref):
    x = _normalize_x(p0_ref, p1_ref)
    x_ref[...] = x
    o_ref[...] = jnp.dot(x, w_ref[...],
                         preferred_element_type=jnp.float32) + b_ref[...]


@functools.lru_cache(maxsize=None)
def _make_d2(n):
    grid = n // BN
    part_spec = pl.BlockSpec((2, BN, 16), lambda i: (0, i, 0))
    return pl.pallas_call(
        _d2_body,
        grid=(grid,),
        in_specs=[
            part_spec, part_spec,
            pl.BlockSpec((32, 64), lambda i: (0, 0)),
            pl.BlockSpec((1, 64), lambda i: (0, 0)),
        ],
        out_specs=[
            pl.BlockSpec((BN, 64), lambda i: (i, 0)),
            pl.BlockSpec((BN, 32), lambda i: (i, 0)),
        ],
        out_shape=[
            jax.ShapeDtypeStruct((n, 64), jnp.float32),
            jax.ShapeDtypeStruct((n, 32), jnp.float32),
        ],
    )


def _d3_body(p0_ref, p1_ref, x1_ref, x2_ref, w_ref, b_ref, bidx_ref,
             o_ref, sacc, cacc, *, grid):
    i = pl.program_id(0)
    x3 = _normalize_x(p0_ref, p1_ref)
    w = w_ref[...]
    x4 = (jnp.dot(x1_ref[...], w[0:32], preferred_element_type=jnp.float32)
          + jnp.dot(x2_ref[...], w[32:64], preferred_element_type=jnp.float32)
          + jnp.dot(x3, w[64:96], preferred_element_type=jnp.float32)
          + b_ref[...])
    b = bidx_ref[0, 0, :]
    oh_t = (lax.broadcasted_iota(jnp.int32, (NUM_GRAPHS, BN), 0)
            == b[None, :]).astype(jnp.float32)

    @pl.when(i == 0)
    def _():
        sacc[...] = jnp.zeros((NUM_GRAPHS, 64), jnp.float32)
        cacc[...] = jnp.zeros((NUM_GRAPHS, 8), jnp.float32)

    sacc[...] += jnp.dot(oh_t, x4, preferred_element_type=jnp.float32)
    cacc[...] += jnp.dot(oh_t, jnp.ones((BN, 8), jnp.float32),
                         preferred_element_type=jnp.float32)

    @pl.when(i == grid - 1)
    def _():
        cnt = jnp.maximum(cacc[:, 0:1], 1.0)
        o_ref[...] = sacc[...] / cnt


@functools.lru_cache(maxsize=None)
def _make_d3(n):
    grid = n // BN
    part_spec = pl.BlockSpec((2, BN, 16), lambda i: (0, i, 0))
    return pl.pallas_call(
        functools.partial(_d3_body, grid=grid),
        grid=(grid,),
        in_specs=[
            part_spec, part_spec,
            pl.BlockSpec((BN, 32), lambda i: (i, 0)),
            pl.BlockSpec((BN, 32), lambda i: (i, 0)),
            pl.BlockSpec((96, 64), lambda i: (0, 0)),
            pl.BlockSpec((1, 64), lambda i: (0, 0)),
            pl.BlockSpec((1, 1, BN), lambda i: (i, 0, 0)),
        ],
        out_specs=pl.BlockSpec((NUM_GRAPHS, 64), lambda i: (0, 0)),
        out_shape=jax.ShapeDtypeStruct((NUM_GRAPHS, 64), jnp.float32),
        scratch_shapes=[
            pltpu.VMEM((NUM_GRAPHS, 64), jnp.float32),
            pltpu.VMEM((NUM_GRAPHS, 8), jnp.float32),
        ],
    )


def _d4_body(m0_ref, m1_ref, m2_ref, w_ref, b_ref, o_ref):
    x = jnp.concatenate([m0_ref[...], m1_ref[...], m2_ref[...]], axis=1)
    y = jnp.dot(x, w_ref[...], preferred_element_type=jnp.float32) + b_ref[...]
    m = jnp.max(y, axis=1, keepdims=True)
    e = jnp.exp(y - m)
    o_ref[...] = e / jnp.sum(e, axis=1, keepdims=True)


@functools.lru_cache(maxsize=None)
def _make_d4():
    return pl.pallas_call(
        _d4_body,
        out_shape=jax.ShapeDtypeStruct((NUM_GRAPHS, 64), jnp.float32),
    )


def _fold_layer(plist, fin_real, fin_pad):
    cols, bs = [], []
    for p in plist:
        u1 = p['W'] @ p['a1_W']
        u2 = p['W'] @ p['a2_W']
        d1 = p['b'] @ p['a1_W'][:, 0] + p['a1_b'][0]
        d2 = p['b'] @ p['a2_W'][:, 0] + p['a2_b'][0]
        cols.append((p['W'], u1, u2, p['b'], d1, d2))
    z1 = jnp.zeros((fin_real, 1), jnp.float32)
    Wc = jnp.concatenate([
        cols[0][0], z1, cols[1][0], z1,
        cols[0][1], cols[0][2], cols[1][1], cols[1][2],
        jnp.zeros((fin_real, 28), jnp.float32)], axis=1)
    if fin_pad > fin_real:
        Wc = jnp.concatenate(
            [Wc, jnp.zeros((fin_pad - fin_real, 64), jnp.float32)], axis=0)
    one = jnp.ones((1,), jnp.float32)
    bc = jnp.concatenate([
        cols[0][3], one, cols[1][3], one,
        jnp.stack([cols[0][4], cols[0][5], cols[1][4], cols[1][5]]),
        jnp.zeros((28,), jnp.float32)])
    return Wc, bc[None, :]


def _pad_w4(W4):
    z2 = jnp.zeros((2, 64), jnp.float32)
    return jnp.concatenate(
        [W4[0:30], z2, W4[30:60], z2, W4[60:90], z2], axis=0)


def _branch(params, pref, X, idx_list, batch):
    n = X.shape[0]
    Wc1, bc1 = _fold_layer(params[pref + '_1'], X.shape[1], X.shape[1])
    Wc2, bc2 = _fold_layer(params[pref + '_2'], 30, 32)
    Wc3, bc3 = _fold_layer(params[pref + '_3'], 30, 32)

    def sc_layer(P):
        parts = []
        for h in range(2):
            E = idx_list[h].shape[1]
            edge = _make_edge(n, E)
            parts.append(edge(idx_list[h][0], idx_list[h][1],
                              P[:, 32 + 2 * h], P[:, 33 + 2 * h],
                              P[:, 16 * h:16 * h + 16]))
        return parts

    P1 = _make_d1(n, X.shape[1])(X, Wc1, bc1)
    parts1 = sc_layer(P1)
    P2, X1 = _make_d2(n)(parts1[0], parts1[1], Wc2, bc2)
    parts2 = sc_layer(P2)
    P3, X2 = _make_d2(n)(parts2[0], parts2[1], Wc3, bc3)
    parts3 = sc_layer(P3)

    batch3d = batch.astype(jnp.int32).reshape(n // BN, 1, BN)
    return _make_d3(n)(parts3[0], parts3[1], X1, X2,
                       _pad_w4(params[pref + '_4W']),
                       params[pref + '_4b'][None, :], batch3d)


def kernel(X0, X1, X2, L0_indices, L0_values, L1u_indices, L1u_values,
           L1d_indices, L1d_values, L2_indices, L2_values,
           batch0, batch1, batch2, params):
    m0 = _branch(params, 'l0', X0, [L0_indices, L0_indices], batch0)
    m1 = _branch(params, 'l1', X1, [L1u_indices, L1d_indices], batch1)
    m2 = _branch(params, 'l2', X2, [L2_indices, L2_indices], batch2)
    return _make_d4()(m0, m1, m2, params['comb_W'], params['comb_b'][None, :])


# speedup vs baseline: 67.6602x; 1.9093x over previous
import functools

import jax
import jax.numpy as jnp
from jax import lax
from jax.experimental import pallas as pl
from jax.experimental.pallas import tpu as pltpu
from jax.experimental.pallas import tpu_sc as plsc

NUM_GRAPHS = 64
CH = 80
ZR = 25
BN = 2000


@functools.lru_cache(maxsize=None)
def _make_edge(n, E):
    per_w = E // 32
    BIG = 5 * CH
    nsub = BIG // CH
    nblocks = per_w // BIG
    nr = n // 16
    assert per_w % BIG == 0 and n % (16 * ZR) == 0

    mesh = plsc.VectorSubcoreMesh(core_axis_name="c", subcore_axis_name="s")

    @functools.partial(
        pl.kernel, mesh=mesh,
        out_type=jax.ShapeDtypeStruct((2, n, 16), jnp.float32),
        compiler_params=pltpu.CompilerParams(needs_layout_passes=False,
                                             use_tc_tiling_on_sc=False),
        scratch_types=[
            pltpu.VMEM((n,), jnp.float32),
            pltpu.VMEM((n,), jnp.float32),
            pltpu.VMEM((BIG,), jnp.int32),
            pltpu.VMEM((BIG,), jnp.int32),
            pltpu.VMEM((CH, 16), jnp.float32),
            pltpu.VMEM((CH, 16), jnp.float32),
            [pltpu.VMEM((CH, 16), jnp.float32) for _ in range(5)],
            [pltpu.VMEM((CH,), jnp.int32) for _ in range(5)],
            pltpu.VMEM((ZR, 16), jnp.float32),
            pltpu.VMEM_SHARED((n, 16), jnp.float32),
            pltpu.SemaphoreType.DMA,
            pltpu.SemaphoreType.DMA,
            pltpu.SemaphoreType.DMA,
            pltpu.SemaphoreType.DMA,
            pltpu.SemaphoreType.DMA,
        ],
    )
    def edge(row_hbm, col_hbm, a1_hbm, a2_hbm, h_hbm, out_hbm,
             a1_t, a2_t, rowB, colB, gath0, gath1, contribs, srows,
             zbuf, acc_sh, sem_ir, sem_ic, sg0, sg1, ss):
        c = lax.axis_index("c")
        s = lax.axis_index("s")
        wid = c * 16 + s
        gath = [gath0, gath1]
        sg = [sg0, sg1]

        pltpu.sync_copy(a1_hbm, a1_t)
        pltpu.sync_copy(a2_hbm, a2_t)

        zero16 = jnp.zeros((16,), jnp.float32)
        for i in range(ZR):
            zbuf[i] = zero16

        def zacc(k, carry):
            pltpu.sync_copy(zbuf, acc_sh.at[pl.ds(s * nr + k * ZR, ZR)])
            return carry
        lax.fori_loop(0, nr // ZR, zacc, 0)
        plsc.subcore_barrier()

        lane = lax.iota(jnp.int32, 16)

        def drain_scatters():
            for j in range(nsub):
                pltpu.make_async_copy(
                    contribs[j], acc_sh.at[srows[j]], ss).wait()

        def block(k, carry):
            base = wid * per_w + k * BIG
            ic_r = pltpu.async_copy(row_hbm.at[pl.ds(base, BIG)], rowB, sem_ir)
            ic_c = pltpu.async_copy(col_hbm.at[pl.ds(base, BIG)], colB, sem_ic)

            @pl.when(k > 0)
            def _():
                drain_scatters()
            ic_r.wait()
            ic_c.wait()

            gcopies = [pltpu.async_copy(
                h_hbm.at[colB.at[pl.ds(0, CH)]], gath[0], sg[0])]
            for j in range(nsub):
                b = j % 2
                gcopies[j].wait()
                if j + 1 < nsub:
                    gcopies.append(pltpu.async_copy(
                        h_hbm.at[colB.at[pl.ds((j + 1) * CH, CH)]],
                        gath[1 - b], sg[1 - b]))
                for g in range(CH // 16):
                    off = j * CH + g * 16
                    r = rowB[pl.ds(off, 16)]
                    cc = colB[pl.ds(off, 16)]
                    srows[j][pl.ds(g * 16, 16)] = r
                    v = (plsc.load_gather(a1_t, [r])
                         + plsc.load_gather(a2_t, [cc]))
                    v = jnp.where(v >= 0.0, v, 0.01 * v)
                    w = jnp.exp(v)
                    ridx = lane + g * 16
                    for f in range(16):
                        fidx = jnp.full((16,), f, jnp.int32)
                        hv = plsc.load_gather(gath[b], [ridx, fidx])
                        plsc.store_scatter(contribs[j], [ridx, fidx], hv * w)
                pltpu.async_copy(contribs[j], acc_sh.at[srows[j]], ss,
                                 add=True)
            return carry
        lax.fori_loop(0, nblocks, block, 0)
        drain_scatters()

        plsc.subcore_barrier()

        @pl.when(s == 0)
        def _():
            pltpu.sync_copy(acc_sh, out_hbm.at[c])

    return edge


def _d1_body(x_ref, w_ref, b_ref, o_ref):
    o_ref[...] = jnp.dot(x_ref[...], w_ref[...],
                         preferred_element_type=jnp.float32) + b_ref[...]


@functools.lru_cache(maxsize=None)
def _make_d1(n, fin):
    grid = n // BN
    return pl.pallas_call(
        _d1_body,
        grid=(grid,),
        in_specs=[
            pl.BlockSpec((BN, fin), lambda i: (i, 0)),
            pl.BlockSpec((fin, 64), lambda i: (0, 0)),
            pl.BlockSpec((1, 64), lambda i: (0, 0)),
        ],
        out_specs=pl.BlockSpec((BN, 64), lambda i: (i, 0)),
        out_shape=jax.ShapeDtypeStruct((n, 64), jnp.float32),
    )


def _normalize_x(p0_ref, p1_ref):
    p0 = p0_ref[0] + p0_ref[1]
    p1 = p1_ref[0] + p1_ref[1]
    s0 = p0[:, 15:16]
    s1 = p1[:, 15:16]
    o0 = p0[:, 0:15] / jnp.where(s0 == 0.0, 1.0, s0)
    o1 = p1[:, 0:15] / jnp.where(s1 == 0.0, 1.0, s1)
    bn = o0.shape[0]
    x = jnp.concatenate([o0, o1, jnp.zeros((bn, 2), jnp.float32)], axis=1)
    return jnp.maximum(x, 0.0)


def _d2_body(p0_ref, p1_ref, w_ref, b_ref, o_ref, x_ref):
    x = _normalize_x(p0_ref, p1_ref)
    x_ref[...] = x
    o_ref[...] = jnp.dot(x, w_ref[...],
                         preferred_element_type=jnp.float32) + b_ref[...]


@functools.lru_cache(maxsize=None)
def _make_d2(n):
    grid = n // BN
    part_spec = pl.BlockSpec((2, BN, 16), lambda i: (0, i, 0))
    return pl.pallas_call(
        _d2_body,
        grid=(grid,),
        in_specs=[
            part_spec, part_spec,
            pl.BlockSpec((32, 64), lambda i: (0, 0)),
            pl.BlockSpec((1, 64), lambda i: (0, 0)),
        ],
        out_specs=[
            pl.BlockSpec((BN, 64), lambda i: (i, 0)),
            pl.BlockSpec((BN, 32), lambda i: (i, 0)),
        ],
        out_shape=[
            jax.ShapeDtypeStruct((n, 64), jnp.float32),
            jax.ShapeDtypeStruct((n, 32), jnp.float32),
        ],
    )


def _d3_body(p0_ref, p1_ref, x1_ref, x2_ref, w_ref, b_ref, bidx_ref,
             o_ref, sacc, cacc, *, grid):
    i = pl.program_id(0)
    x3 = _normalize_x(p0_ref, p1_ref)
    w = w_ref[...]
    x4 = (jnp.dot(x1_ref[...], w[0:32], preferred_element_type=jnp.float32)
          + jnp.dot(x2_ref[...], w[32:64], preferred_element_type=jnp.float32)
          + jnp.dot(x3, w[64:96], preferred_element_type=jnp.float32)
          + b_ref[...])
    b = bidx_ref[0, 0, :]
    oh_t = (lax.broadcasted_iota(jnp.int32, (NUM_GRAPHS, BN), 0)
            == b[None, :]).astype(jnp.float32)

    @pl.when(i == 0)
    def _():
        sacc[...] = jnp.zeros((NUM_GRAPHS, 64), jnp.float32)
        cacc[...] = jnp.zeros((NUM_GRAPHS, 8), jnp.float32)

    sacc[...] += jnp.dot(oh_t, x4, preferred_element_type=jnp.float32)
    cacc[...] += jnp.dot(oh_t, jnp.ones((BN, 8), jnp.float32),
                         preferred_element_type=jnp.float32)

    @pl.when(i == grid - 1)
    def _():
        cnt = jnp.maximum(cacc[:, 0:1], 1.0)
        o_ref[...] = sacc[...] / cnt


@functools.lru_cache(maxsize=None)
def _make_d3(n):
    grid = n // BN
    part_spec = pl.BlockSpec((2, BN, 16), lambda i: (0, i, 0))
    return pl.pallas_call(
        functools.partial(_d3_body, grid=grid),
        grid=(grid,),
        in_specs=[
            part_spec, part_spec,
            pl.BlockSpec((BN, 32), lambda i: (i, 0)),
            pl.BlockSpec((BN, 32), lambda i: (i, 0)),
            pl.BlockSpec((96, 64), lambda i: (0, 0)),
            pl.BlockSpec((1, 64), lambda i: (0, 0)),
            pl.BlockSpec((1, 1, BN), lambda i: (i, 0, 0)),
        ],
        out_specs=pl.BlockSpec((NUM_GRAPHS, 64), lambda i: (0, 0)),
        out_shape=jax.ShapeDtypeStruct((NUM_GRAPHS, 64), jnp.float32),
        scratch_shapes=[
            pltpu.VMEM((NUM_GRAPHS, 64), jnp.float32),
            pltpu.VMEM((NUM_GRAPHS, 8), jnp.float32),
        ],
    )


def _d4_body(m0_ref, m1_ref, m2_ref, w_ref, b_ref, o_ref):
    x = jnp.concatenate([m0_ref[...], m1_ref[...], m2_ref[...]], axis=1)
    y = jnp.dot(x, w_ref[...], preferred_element_type=jnp.float32) + b_ref[...]
    m = jnp.max(y, axis=1, keepdims=True)
    e = jnp.exp(y - m)
    o_ref[...] = e / jnp.sum(e, axis=1, keepdims=True)


@functools.lru_cache(maxsize=None)
def _make_d4():
    return pl.pallas_call(
        _d4_body,
        out_shape=jax.ShapeDtypeStruct((NUM_GRAPHS, 64), jnp.float32),
    )


def _fold_layer(plist, fin_real, fin_pad):
    cols, bs = [], []
    for p in plist:
        u1 = p['W'] @ p['a1_W']
        u2 = p['W'] @ p['a2_W']
        d1 = p['b'] @ p['a1_W'][:, 0] + p['a1_b'][0]
        d2 = p['b'] @ p['a2_W'][:, 0] + p['a2_b'][0]
        cols.append((p['W'], u1, u2, p['b'], d1, d2))
    z1 = jnp.zeros((fin_real, 1), jnp.float32)
    Wc = jnp.concatenate([
        cols[0][0], z1, cols[1][0], z1,
        cols[0][1], cols[0][2], cols[1][1], cols[1][2],
        jnp.zeros((fin_real, 28), jnp.float32)], axis=1)
    if fin_pad > fin_real:
        Wc = jnp.concatenate(
            [Wc, jnp.zeros((fin_pad - fin_real, 64), jnp.float32)], axis=0)
    one = jnp.ones((1,), jnp.float32)
    bc = jnp.concatenate([
        cols[0][3], one, cols[1][3], one,
        jnp.stack([cols[0][4], cols[0][5], cols[1][4], cols[1][5]]),
        jnp.zeros((28,), jnp.float32)])
    return Wc, bc[None, :]


def _pad_w4(W4):
    z2 = jnp.zeros((2, 64), jnp.float32)
    return jnp.concatenate(
        [W4[0:30], z2, W4[30:60], z2, W4[60:90], z2], axis=0)


def _branch(params, pref, X, idx_list, batch):
    n = X.shape[0]
    Wc1, bc1 = _fold_layer(params[pref + '_1'], X.shape[1], X.shape[1])
    Wc2, bc2 = _fold_layer(params[pref + '_2'], 30, 32)
    Wc3, bc3 = _fold_layer(params[pref + '_3'], 30, 32)

    def sc_layer(P):
        parts = []
        for h in range(2):
            E = idx_list[h].shape[1]
            edge = _make_edge(n, E)
            parts.append(edge(idx_list[h][0], idx_list[h][1],
                              P[:, 32 + 2 * h], P[:, 33 + 2 * h],
                              P[:, 16 * h:16 * h + 16]))
        return parts

    P1 = _make_d1(n, X.shape[1])(X, Wc1, bc1)
    parts1 = sc_layer(P1)
    P2, X1 = _make_d2(n)(parts1[0], parts1[1], Wc2, bc2)
    parts2 = sc_layer(P2)
    P3, X2 = _make_d2(n)(parts2[0], parts2[1], Wc3, bc3)
    parts3 = sc_layer(P3)

    batch3d = batch.astype(jnp.int32).reshape(n // BN, 1, BN)
    return _make_d3(n)(parts3[0], parts3[1], X1, X2,
                       _pad_w4(params[pref + '_4W']),
                       params[pref + '_4b'][None, :], batch3d)


def kernel(X0, X1, X2, L0_indices, L0_values, L1u_indices, L1u_values,
           L1d_indices, L1d_values, L2_indices, L2_values,
           batch0, batch1, batch2, params):
    m0 = _branch(params, 'l0', X0, [L0_indices, L0_indices], batch0)
    m1 = _branch(params, 'l1', X1, [L1u_indices, L1d_indices], batch1)
    m2 = _branch(params, 'l2', X2, [L2_indices, L2_indices], batch2)
    return _make_d4()(m0, m1, m2, params['comb_W'], params['comb_b'][None, :])


# speedup vs baseline: 84.8879x; 1.2546x over previous
import functools

import jax
import jax.numpy as jnp
from jax import lax
from jax.experimental import pallas as pl
from jax.experimental.pallas import tpu as pltpu
from jax.experimental.pallas import tpu_sc as plsc

NUM_GRAPHS = 64
CH = 80
ZR = 25
BN = 2000


@functools.lru_cache(maxsize=None)
def _make_edge(n, E):
    per_w = E // 32
    BIG = 5 * CH
    nsub = BIG // CH
    nblocks = per_w // BIG
    nr = n // 16
    assert per_w % BIG == 0 and n % (16 * ZR) == 0

    mesh = plsc.VectorSubcoreMesh(core_axis_name="c", subcore_axis_name="s")

    @functools.partial(
        pl.kernel, mesh=mesh,
        out_type=jax.ShapeDtypeStruct((2, n, 16), jnp.float32),
        compiler_params=pltpu.CompilerParams(needs_layout_passes=False,
                                             use_tc_tiling_on_sc=False),
        scratch_types=[
            pltpu.VMEM((n,), jnp.float32),
            pltpu.VMEM((n,), jnp.float32),
            pltpu.VMEM((BIG,), jnp.int32),
            pltpu.VMEM((BIG,), jnp.int32),
            pltpu.VMEM((CH, 16), jnp.float32),
            pltpu.VMEM((CH, 16), jnp.float32),
            [pltpu.VMEM((CH, 16), jnp.float32) for _ in range(5)],
            [pltpu.VMEM((CH,), jnp.int32) for _ in range(5)],
            pltpu.VMEM((CH,), jnp.float32),
            pltpu.VMEM((ZR, 16), jnp.float32),
            pltpu.VMEM_SHARED((n, 16), jnp.float32),
            pltpu.SemaphoreType.DMA,
            pltpu.SemaphoreType.DMA,
            pltpu.SemaphoreType.DMA,
            pltpu.SemaphoreType.DMA,
            pltpu.SemaphoreType.DMA,
        ],
    )
    def edge(row_hbm, col_hbm, a1_hbm, a2_hbm, h_hbm, out_hbm,
             a1_t, a2_t, rowB, colB, gath0, gath1, contribs, srows,
             wbuf, zbuf, acc_sh, sem_ir, sem_ic, sg0, sg1, ss):
        c = lax.axis_index("c")
        s = lax.axis_index("s")
        wid = c * 16 + s
        gath = [gath0, gath1]
        sg = [sg0, sg1]

        pltpu.sync_copy(a1_hbm, a1_t)
        pltpu.sync_copy(a2_hbm, a2_t)

        zero16 = jnp.zeros((16,), jnp.float32)
        for i in range(ZR):
            zbuf[i] = zero16

        def zacc(k, carry):
            pltpu.sync_copy(zbuf, acc_sh.at[pl.ds(s * nr + k * ZR, ZR)])
            return carry
        lax.fori_loop(0, nr // ZR, zacc, 0)
        plsc.subcore_barrier()

        lane = lax.iota(jnp.int32, 16)

        def drain_scatters():
            for j in range(nsub):
                pltpu.make_async_copy(
                    contribs[j], acc_sh.at[srows[j]], ss).wait()

        def block(k, carry):
            base = wid * per_w + k * BIG
            ic_r = pltpu.async_copy(row_hbm.at[pl.ds(base, BIG)], rowB, sem_ir)
            ic_c = pltpu.async_copy(col_hbm.at[pl.ds(base, BIG)], colB, sem_ic)

            @pl.when(k > 0)
            def _():
                drain_scatters()
            ic_r.wait()
            ic_c.wait()

            gcopies = [pltpu.async_copy(
                h_hbm.at[colB.at[pl.ds(0, CH)]], gath[0], sg[0])]
            for j in range(nsub):
                b = j % 2
                gcopies[j].wait()
                if j + 1 < nsub:
                    gcopies.append(pltpu.async_copy(
                        h_hbm.at[colB.at[pl.ds((j + 1) * CH, CH)]],
                        gath[1 - b], sg[1 - b]))
                for g in range(CH // 16):
                    off = j * CH + g * 16
                    r = rowB[pl.ds(off, 16)]
                    cc = colB[pl.ds(off, 16)]
                    srows[j][pl.ds(g * 16, 16)] = r
                    v = (plsc.load_gather(a1_t, [r])
                         + plsc.load_gather(a2_t, [cc]))
                    v = jnp.where(v >= 0.0, v, 0.01 * v)
                    w = jnp.exp(v)
                    for eloc in range(16):
                        e = g * 16 + eloc
                        contribs[j][e] = gath[b][e] * w[eloc]
                pltpu.async_copy(contribs[j], acc_sh.at[srows[j]], ss,
                                 add=True)
            return carry
        lax.fori_loop(0, nblocks, block, 0)
        drain_scatters()

        plsc.subcore_barrier()

        @pl.when(s == 0)
        def _():
            pltpu.sync_copy(acc_sh, out_hbm.at[c])

    return edge


def _d1_body(x_ref, w_ref, b_ref, o_ref):
    o_ref[...] = jnp.dot(x_ref[...], w_ref[...],
                         preferred_element_type=jnp.float32) + b_ref[...]


@functools.lru_cache(maxsize=None)
def _make_d1(n, fin):
    grid = n // BN
    return pl.pallas_call(
        _d1_body,
        grid=(grid,),
        in_specs=[
            pl.BlockSpec((BN, fin), lambda i: (i, 0)),
            pl.BlockSpec((fin, 64), lambda i: (0, 0)),
            pl.BlockSpec((1, 64), lambda i: (0, 0)),
        ],
        out_specs=pl.BlockSpec((BN, 64), lambda i: (i, 0)),
        out_shape=jax.ShapeDtypeStruct((n, 64), jnp.float32),
    )


def _normalize_x(p0_ref, p1_ref):
    p0 = p0_ref[0] + p0_ref[1]
    p1 = p1_ref[0] + p1_ref[1]
    s0 = p0[:, 15:16]
    s1 = p1[:, 15:16]
    o0 = p0[:, 0:15] / jnp.where(s0 == 0.0, 1.0, s0)
    o1 = p1[:, 0:15] / jnp.where(s1 == 0.0, 1.0, s1)
    bn = o0.shape[0]
    x = jnp.concatenate([o0, o1, jnp.zeros((bn, 2), jnp.float32)], axis=1)
    return jnp.maximum(x, 0.0)


def _d2_body(p0_ref, p1_ref, w_ref, b_ref, o_ref, x_ref):
    x = _normalize_x(p0_ref, p1_ref)
    x_ref[...] = x
    o_ref[...] = jnp.dot(x, w_ref[...],
                         preferred_element_type=jnp.float32) + b_ref[...]


@functools.lru_cache(maxsize=None)
def _make_d2(n):
    grid = n // BN
    part_spec = pl.BlockSpec((2, BN, 16), lambda i: (0, i, 0))
    return pl.pallas_call(
        _d2_body,
        grid=(grid,),
        in_specs=[
            part_spec, part_spec,
            pl.BlockSpec((32, 64), lambda i: (0, 0)),
            pl.BlockSpec((1, 64), lambda i: (0, 0)),
        ],
        out_specs=[
            pl.BlockSpec((BN, 64), lambda i: (i, 0)),
            pl.BlockSpec((BN, 32), lambda i: (i, 0)),
        ],
        out_shape=[
            jax.ShapeDtypeStruct((n, 64), jnp.float32),
            jax.ShapeDtypeStruct((n, 32), jnp.float32),
        ],
    )


def _d3_body(p0_ref, p1_ref, x1_ref, x2_ref, w_ref, b_ref, bidx_ref,
             o_ref, sacc, cacc, *, grid):
    i = pl.program_id(0)
    x3 = _normalize_x(p0_ref, p1_ref)
    w = w_ref[...]
    x4 = (jnp.dot(x1_ref[...], w[0:32], preferred_element_type=jnp.float32)
          + jnp.dot(x2_ref[...], w[32:64], preferred_element_type=jnp.float32)
          + jnp.dot(x3, w[64:96], preferred_element_type=jnp.float32)
          + b_ref[...])
    b = bidx_ref[0, 0, :]
    oh_t = (lax.broadcasted_iota(jnp.int32, (NUM_GRAPHS, BN), 0)
            == b[None, :]).astype(jnp.float32)

    @pl.when(i == 0)
    def _():
        sacc[...] = jnp.zeros((NUM_GRAPHS, 64), jnp.float32)
        cacc[...] = jnp.zeros((NUM_GRAPHS, 8), jnp.float32)

    sacc[...] += jnp.dot(oh_t, x4, preferred_element_type=jnp.float32)
    cacc[...] += jnp.dot(oh_t, jnp.ones((BN, 8), jnp.float32),
                         preferred_element_type=jnp.float32)

    @pl.when(i == grid - 1)
    def _():
        cnt = jnp.maximum(cacc[:, 0:1], 1.0)
        o_ref[...] = sacc[...] / cnt


@functools.lru_cache(maxsize=None)
def _make_d3(n):
    grid = n // BN
    part_spec = pl.BlockSpec((2, BN, 16), lambda i: (0, i, 0))
    return pl.pallas_call(
        functools.partial(_d3_body, grid=grid),
        grid=(grid,),
        in_specs=[
            part_spec, part_spec,
            pl.BlockSpec((BN, 32), lambda i: (i, 0)),
            pl.BlockSpec((BN, 32), lambda i: (i, 0)),
            pl.BlockSpec((96, 64), lambda i: (0, 0)),
            pl.BlockSpec((1, 64), lambda i: (0, 0)),
            pl.BlockSpec((1, 1, BN), lambda i: (i, 0, 0)),
        ],
        out_specs=pl.BlockSpec((NUM_GRAPHS, 64), lambda i: (0, 0)),
        out_shape=jax.ShapeDtypeStruct((NUM_GRAPHS, 64), jnp.float32),
        scratch_shapes=[
            pltpu.VMEM((NUM_GRAPHS, 64), jnp.float32),
            pltpu.VMEM((NUM_GRAPHS, 8), jnp.float32),
        ],
    )


def _d4_body(m0_ref, m1_ref, m2_ref, w_ref, b_ref, o_ref):
    x = jnp.concatenate([m0_ref[...], m1_ref[...], m2_ref[...]], axis=1)
    y = jnp.dot(x, w_ref[...], preferred_element_type=jnp.float32) + b_ref[...]
    m = jnp.max(y, axis=1, keepdims=True)
    e = jnp.exp(y - m)
    o_ref[...] = e / jnp.sum(e, axis=1, keepdims=True)


@functools.lru_cache(maxsize=None)
def _make_d4():
    return pl.pallas_call(
        _d4_body,
        out_shape=jax.ShapeDtypeStruct((NUM_GRAPHS, 64), jnp.float32),
    )


def _fold_layer(plist, fin_real, fin_pad):
    cols, bs = [], []
    for p in plist:
        u1 = p['W'] @ p['a1_W']
        u2 = p['W'] @ p['a2_W']
        d1 = p['b'] @ p['a1_W'][:, 0] + p['a1_b'][0]
        d2 = p['b'] @ p['a2_W'][:, 0] + p['a2_b'][0]
        cols.append((p['W'], u1, u2, p['b'], d1, d2))
    z1 = jnp.zeros((fin_real, 1), jnp.float32)
    Wc = jnp.concatenate([
        cols[0][0], z1, cols[1][0], z1,
        cols[0][1], cols[0][2], cols[1][1], cols[1][2],
        jnp.zeros((fin_real, 28), jnp.float32)], axis=1)
    if fin_pad > fin_real:
        Wc = jnp.concatenate(
            [Wc, jnp.zeros((fin_pad - fin_real, 64), jnp.float32)], axis=0)
    one = jnp.ones((1,), jnp.float32)
    bc = jnp.concatenate([
        cols[0][3], one, cols[1][3], one,
        jnp.stack([cols[0][4], cols[0][5], cols[1][4], cols[1][5]]),
        jnp.zeros((28,), jnp.float32)])
    return Wc, bc[None, :]


def _pad_w4(W4):
    z2 = jnp.zeros((2, 64), jnp.float32)
    return jnp.concatenate(
        [W4[0:30], z2, W4[30:60], z2, W4[60:90], z2], axis=0)


def _branch(params, pref, X, idx_list, batch):
    n = X.shape[0]
    Wc1, bc1 = _fold_layer(params[pref + '_1'], X.shape[1], X.shape[1])
    Wc2, bc2 = _fold_layer(params[pref + '_2'], 30, 32)
    Wc3, bc3 = _fold_layer(params[pref + '_3'], 30, 32)

    def sc_layer(P):
        parts = []
        for h in range(2):
            E = idx_list[h].shape[1]
            edge = _make_edge(n, E)
            parts.append(edge(idx_list[h][0], idx_list[h][1],
                              P[:, 32 + 2 * h], P[:, 33 + 2 * h],
                              P[:, 16 * h:16 * h + 16]))
        return parts

    P1 = _make_d1(n, X.shape[1])(X, Wc1, bc1)
    parts1 = sc_layer(P1)
    P2, X1 = _make_d2(n)(parts1[0], parts1[1], Wc2, bc2)
    parts2 = sc_layer(P2)
    P3, X2 = _make_d2(n)(parts2[0], parts2[1], Wc3, bc3)
    parts3 = sc_layer(P3)

    batch3d = batch.astype(jnp.int32).reshape(n // BN, 1, BN)
    return _make_d3(n)(parts3[0], parts3[1], X1, X2,
                       _pad_w4(params[pref + '_4W']),
                       params[pref + '_4b'][None, :], batch3d)


def kernel(X0, X1, X2, L0_indices, L0_values, L1u_indices, L1u_values,
           L1d_indices, L1d_values, L2_indices, L2_values,
           batch0, batch1, batch2, params):
    m0 = _branch(params, 'l0', X0, [L0_indices, L0_indices], batch0)
    m1 = _branch(params, 'l1', X1, [L1u_indices, L1d_indices], batch1)
    m2 = _branch(params, 'l2', X2, [L2_indices, L2_indices], batch2)
    return _make_d4()(m0, m1, m2, params['comb_W'], params['comb_b'][None, :])


# speedup vs baseline: 84.9313x; 1.0005x over previous
import functools

import jax
import jax.numpy as jnp
from jax import lax
from jax.experimental import pallas as pl
from jax.experimental.pallas import tpu as pltpu
from jax.experimental.pallas import tpu_sc as plsc

NUM_GRAPHS = 64
CH = 80
ZR = 25
BN = 2000


@functools.lru_cache(maxsize=None)
def _make_edge(n, E):
    per_w = E // 32
    BIG = 5 * CH
    nsub = BIG // CH
    nblocks = per_w // BIG
    nr = n // 16
    assert per_w % BIG == 0 and n % (16 * ZR) == 0

    mesh = plsc.VectorSubcoreMesh(core_axis_name="c", subcore_axis_name="s")

    @functools.partial(
        pl.kernel, mesh=mesh,
        out_type=jax.ShapeDtypeStruct((2, n, 16), jnp.float32),
        compiler_params=pltpu.CompilerParams(needs_layout_passes=False,
                                             use_tc_tiling_on_sc=False),
        scratch_types=[
            pltpu.VMEM((n,), jnp.float32),
            pltpu.VMEM((n,), jnp.float32),
            pltpu.VMEM((BIG,), jnp.int32),
            pltpu.VMEM((BIG,), jnp.int32),
            pltpu.VMEM((CH, 16), jnp.float32),
            pltpu.VMEM((CH, 16), jnp.float32),
            [pltpu.VMEM((CH, 16), jnp.float32) for _ in range(5)],
            [pltpu.VMEM((CH,), jnp.int32) for _ in range(5)],
            pltpu.VMEM((ZR, 16), jnp.float32),
            pltpu.VMEM_SHARED((n, 16), jnp.float32),
            pltpu.SemaphoreType.DMA,
            pltpu.SemaphoreType.DMA,
            pltpu.SemaphoreType.DMA,
            pltpu.SemaphoreType.DMA,
            pltpu.SemaphoreType.DMA,
        ],
    )
    def edge(row_hbm, col_hbm, a1_hbm, a2_hbm, h_hbm, out_hbm,
             a1_t, a2_t, rowB, colB, gath0, gath1, contribs, srows,
             zbuf, acc_sh, sem_ir, sem_ic, sg0, sg1, ss):
        c = lax.axis_index("c")
        s = lax.axis_index("s")
        wid = c * 16 + s
        gath = [gath0, gath1]
        sg = [sg0, sg1]

        pltpu.sync_copy(a1_hbm, a1_t)
        pltpu.sync_copy(a2_hbm, a2_t)

        zero16 = jnp.zeros((16,), jnp.float32)
        for i in range(ZR):
            zbuf[i] = zero16

        def zacc(k, carry):
            pltpu.sync_copy(zbuf, acc_sh.at[pl.ds(s * nr + k * ZR, ZR)])
            return carry
        lax.fori_loop(0, nr // ZR, zacc, 0)
        plsc.subcore_barrier()

        def drain_scatters():
            for j in range(nsub):
                pltpu.make_async_copy(
                    contribs[j], acc_sh.at[srows[j]], ss).wait()

        def block(k, carry):
            base = wid * per_w + k * BIG
            ic_r = pltpu.async_copy(row_hbm.at[pl.ds(base, BIG)], rowB, sem_ir)
            ic_c = pltpu.async_copy(col_hbm.at[pl.ds(base, BIG)], colB, sem_ic)

            @pl.when(k > 0)
            def _():
                drain_scatters()
            ic_r.wait()
            ic_c.wait()

            gcopies = [pltpu.async_copy(
                h_hbm.at[colB.at[pl.ds(0, CH)]], gath[0], sg[0])]
            for j in range(nsub):
                b = j % 2
                gcopies[j].wait()
                if j + 1 < nsub:
                    gcopies.append(pltpu.async_copy(
                        h_hbm.at[colB.at[pl.ds((j + 1) * CH, CH)]],
                        gath[1 - b], sg[1 - b]))
                for g in range(CH // 16):
                    off = j * CH + g * 16
                    r = rowB[pl.ds(off, 16)]
                    cc = colB[pl.ds(off, 16)]
                    srows[j][pl.ds(g * 16, 16)] = r
                    v = (plsc.load_gather(a1_t, [r])
                         + plsc.load_gather(a2_t, [cc]))
                    v = jnp.where(v >= 0.0, v, 0.01 * v)
                    w = jnp.exp(v)
                    for eloc in range(16):
                        e = g * 16 + eloc
                        contribs[j][e] = gath[b][e] * w[eloc]
                pltpu.async_copy(contribs[j], acc_sh.at[srows[j]], ss,
                                 add=True)
            return carry
        lax.fori_loop(0, nblocks, block, 0)
        drain_scatters()

        plsc.subcore_barrier()

        @pl.when(s == 0)
        def _():
            pltpu.sync_copy(acc_sh, out_hbm.at[c])

    return edge


def _d1_body(x_ref, w_ref, b_ref, o_ref):
    o_ref[...] = jnp.dot(x_ref[...], w_ref[...],
                         preferred_element_type=jnp.float32) + b_ref[...]


@functools.lru_cache(maxsize=None)
def _make_d1(n, fin):
    grid = n // BN
    return pl.pallas_call(
        _d1_body,
        grid=(grid,),
        in_specs=[
            pl.BlockSpec((BN, fin), lambda i: (i, 0)),
            pl.BlockSpec((fin, 64), lambda i: (0, 0)),
            pl.BlockSpec((1, 64), lambda i: (0, 0)),
        ],
        out_specs=pl.BlockSpec((BN, 64), lambda i: (i, 0)),
        out_shape=jax.ShapeDtypeStruct((n, 64), jnp.float32),
    )


def _normalize_x(p0_ref, p1_ref):
    p0 = p0_ref[0] + p0_ref[1]
    p1 = p1_ref[0] + p1_ref[1]
    s0 = p0[:, 15:16]
    s1 = p1[:, 15:16]
    o0 = p0[:, 0:15] / jnp.where(s0 == 0.0, 1.0, s0)
    o1 = p1[:, 0:15] / jnp.where(s1 == 0.0, 1.0, s1)
    bn = o0.shape[0]
    x = jnp.concatenate([o0, o1, jnp.zeros((bn, 2), jnp.float32)], axis=1)
    return jnp.maximum(x, 0.0)


def _d2_body(p0_ref, p1_ref, w_ref, b_ref, o_ref, x_ref):
    x = _normalize_x(p0_ref, p1_ref)
    x_ref[...] = x
    o_ref[...] = jnp.dot(x, w_ref[...],
                         preferred_element_type=jnp.float32) + b_ref[...]


@functools.lru_cache(maxsize=None)
def _make_d2(n):
    grid = n // BN
    part_spec = pl.BlockSpec((2, BN, 16), lambda i: (0, i, 0))
    return pl.pallas_call(
        _d2_body,
        grid=(grid,),
        in_specs=[
            part_spec, part_spec,
            pl.BlockSpec((32, 64), lambda i: (0, 0)),
            pl.BlockSpec((1, 64), lambda i: (0, 0)),
        ],
        out_specs=[
            pl.BlockSpec((BN, 64), lambda i: (i, 0)),
            pl.BlockSpec((BN, 32), lambda i: (i, 0)),
        ],
        out_shape=[
            jax.ShapeDtypeStruct((n, 64), jnp.float32),
            jax.ShapeDtypeStruct((n, 32), jnp.float32),
        ],
    )


def _d3_body(p0_ref, p1_ref, x1_ref, x2_ref, w_ref, b_ref, bidx_ref,
             o_ref, sacc, cacc, *, grid):
    i = pl.program_id(0)
    x3 = _normalize_x(p0_ref, p1_ref)
    w = w_ref[...]
    x4 = (jnp.dot(x1_ref[...], w[0:32], preferred_element_type=jnp.float32)
          + jnp.dot(x2_ref[...], w[32:64], preferred_element_type=jnp.float32)
          + jnp.dot(x3, w[64:96], preferred_element_type=jnp.float32)
          + b_ref[...])
    b = bidx_ref[0, 0, :]
    oh_t = (lax.broadcasted_iota(jnp.int32, (NUM_GRAPHS, BN), 0)
            == b[None, :]).astype(jnp.float32)

    @pl.when(i == 0)
    def _():
        sacc[...] = jnp.zeros((NUM_GRAPHS, 64), jnp.float32)
        cacc[...] = jnp.zeros((NUM_GRAPHS, 8), jnp.float32)

    sacc[...] += jnp.dot(oh_t, x4, preferred_element_type=jnp.float32)
    cacc[...] += jnp.dot(oh_t, jnp.ones((BN, 8), jnp.float32),
                         preferred_element_type=jnp.float32)

    @pl.when(i == grid - 1)
    def _():
        cnt = jnp.maximum(cacc[:, 0:1], 1.0)
        o_ref[...] = sacc[...] / cnt


@functools.lru_cache(maxsize=None)
def _make_d3(n):
    grid = n // BN
    part_spec = pl.BlockSpec((2, BN, 16), lambda i: (0, i, 0))
    return pl.pallas_call(
        functools.partial(_d3_body, grid=grid),
        grid=(grid,),
        in_specs=[
            part_spec, part_spec,
            pl.BlockSpec((BN, 32), lambda i: (i, 0)),
            pl.BlockSpec((BN, 32), lambda i: (i, 0)),
            pl.BlockSpec((96, 64), lambda i: (0, 0)),
            pl.BlockSpec((1, 64), lambda i: (0, 0)),
            pl.BlockSpec((1, 1, BN), lambda i: (i, 0, 0)),
        ],
        out_specs=pl.BlockSpec((NUM_GRAPHS, 64), lambda i: (0, 0)),
        out_shape=jax.ShapeDtypeStruct((NUM_GRAPHS, 64), jnp.float32),
        scratch_shapes=[
            pltpu.VMEM((NUM_GRAPHS, 64), jnp.float32),
            pltpu.VMEM((NUM_GRAPHS, 8), jnp.float32),
        ],
    )


def _d4_body(m0_ref, m1_ref, m2_ref, w_ref, b_ref, o_ref):
    x = jnp.concatenate([m0_ref[...], m1_ref[...], m2_ref[...]], axis=1)
    y = jnp.dot(x, w_ref[...], preferred_element_type=jnp.float32) + b_ref[...]
    m = jnp.max(y, axis=1, keepdims=True)
    e = jnp.exp(y - m)
    o_ref[...] = e / jnp.sum(e, axis=1, keepdims=True)


@functools.lru_cache(maxsize=None)
def _make_d4():
    return pl.pallas_call(
        _d4_body,
        out_shape=jax.ShapeDtypeStruct((NUM_GRAPHS, 64), jnp.float32),
    )


def _fold_layer(plist, fin_real, fin_pad):
    cols, bs = [], []
    for p in plist:
        u1 = p['W'] @ p['a1_W']
        u2 = p['W'] @ p['a2_W']
        d1 = p['b'] @ p['a1_W'][:, 0] + p['a1_b'][0]
        d2 = p['b'] @ p['a2_W'][:, 0] + p['a2_b'][0]
        cols.append((p['W'], u1, u2, p['b'], d1, d2))
    z1 = jnp.zeros((fin_real, 1), jnp.float32)
    Wc = jnp.concatenate([
        cols[0][0], z1, cols[1][0], z1,
        cols[0][1], cols[0][2], cols[1][1], cols[1][2],
        jnp.zeros((fin_real, 28), jnp.float32)], axis=1)
    if fin_pad > fin_real:
        Wc = jnp.concatenate(
            [Wc, jnp.zeros((fin_pad - fin_real, 64), jnp.float32)], axis=0)
    one = jnp.ones((1,), jnp.float32)
    bc = jnp.concatenate([
        cols[0][3], one, cols[1][3], one,
        jnp.stack([cols[0][4], cols[0][5], cols[1][4], cols[1][5]]),
        jnp.zeros((28,), jnp.float32)])
    return Wc, bc[None, :]


def _pad_w4(W4):
    z2 = jnp.zeros((2, 64), jnp.float32)
    return jnp.concatenate(
        [W4[0:30], z2, W4[30:60], z2, W4[60:90], z2], axis=0)


def _branch(params, pref, X, idx_list, batch):
    n = X.shape[0]
    Wc1, bc1 = _fold_layer(params[pref + '_1'], X.shape[1], X.shape[1])
    Wc2, bc2 = _fold_layer(params[pref + '_2'], 30, 32)
    Wc3, bc3 = _fold_layer(params[pref + '_3'], 30, 32)

    def sc_layer(P):
        parts = []
        for h in range(2):
            E = idx_list[h].shape[1]
            edge = _make_edge(n, E)
            parts.append(edge(idx_list[h][0], idx_list[h][1],
                              P[:, 32 + 2 * h], P[:, 33 + 2 * h],
                              P[:, 16 * h:16 * h + 16]))
        return parts

    P1 = _make_d1(n, X.shape[1])(X, Wc1, bc1)
    parts1 = sc_layer(P1)
    P2, X1 = _make_d2(n)(parts1[0], parts1[1], Wc2, bc2)
    parts2 = sc_layer(P2)
    P3, X2 = _make_d2(n)(parts2[0], parts2[1], Wc3, bc3)
    parts3 = sc_layer(P3)

    batch3d = batch.astype(jnp.int32).reshape(n // BN, 1, BN)
    return _make_d3(n)(parts3[0], parts3[1], X1, X2,
                       _pad_w4(params[pref + '_4W']),
                       params[pref + '_4b'][None, :], batch3d)


def kernel(X0, X1, X2, L0_indices, L0_values, L1u_indices, L1u_values,
           L1d_indices, L1d_values, L2_indices, L2_values,
           batch0, batch1, batch2, params):
    m0 = _branch(params, 'l0', X0, [L0_indices, L0_indices], batch0)
    m1 = _branch(params, 'l1', X1, [L1u_indices, L1d_indices], batch1)
    m2 = _branch(params, 'l2', X2, [L2_indices, L2_indices], batch2)
    return _make_d4()(m0, m1, m2, params['comb_W'], params['comb_b'][None, :])
